# baseline (speedup 1.0000x reference)
import functools

import jax
import jax.numpy as jnp
from jax import lax
from jax.experimental import pallas as pl
from jax.experimental.pallas import tpu as pltpu
from jax.experimental.pallas import tpu_sc as plsc

N = 10000
D = 128
E = 320000
BN_EPS = 1e-5

NC = 2
NS = 16
NW = NC * NS
EPW = E // NW
W = 40
F = EPW // W
NB = 6
SS = F // NB
TAIL = F - SS * NB
NP = 10240
RPS = NP // NS
L = 16


def _sc_aggregate(h, src, dst):
  mesh = plsc.VectorSubcoreMesh(core_axis_name="c", subcore_axis_name="s")

  @functools.partial(
      pl.kernel,
      out_type=(
          jax.ShapeDtypeStruct((NC * NP, D), jnp.float32),
          jax.ShapeDtypeStruct((NC * NP, D), jnp.float32),
      ),
      mesh=mesh,
      scratch_types=(
          [pltpu.VMEM((W,), jnp.int32) for _ in range(NB)]
          + [pltpu.VMEM((W,), jnp.int32) for _ in range(NB)]
          + [pltpu.VMEM((W, D), jnp.float32) for _ in range(NB)]
          + [pltpu.SemaphoreType.DMA for _ in range(3 * NB)]
          + [pltpu.VMEM_SHARED((NP, D), jnp.float32)]
      ),
  )
  def agg_kernel(h_hbm, src_hbm, dst_hbm, p_hbm, c_hbm, *bufs):
    srcv = bufs[0:NB]
    dstv = bufs[NB:2 * NB]
    rows = bufs[2 * NB:3 * NB]
    isem = bufs[3 * NB:4 * NB]
    gsem = bufs[4 * NB:5 * NB]
    ssem = bufs[5 * NB:6 * NB]
    acc_sh = bufs[6 * NB]

    cid = lax.axis_index("c")
    sid = lax.axis_index("s")
    wid = cid * NS + sid
    ebase = wid * EPW
    row0 = sid * RPS
    obase = cid * NP + row0

    def fill(ref, val16):
      @pl.loop(0, W)
      def _(r):
        for cc in range(D // L):
          ref[r, pl.ds(cc * L, L)] = val16

    def zero_my_slice():
      for t in range(RPS // W):
        pltpu.sync_copy(rows[1], acc_sh.at[pl.ds(row0 + t * W, W)])

    z16 = jnp.zeros((L,), jnp.float32)
    o16 = jnp.ones((L,), jnp.float32)

    fill(rows[1], z16)
    zero_my_slice()
    plsc.subcore_barrier()

    @pl.loop(0, SS)
    def _(s):
      base = ebase + s * NB * W
      ih = []
      for j in range(NB):
        bj = base + j * W
        h1 = pltpu.async_copy(src_hbm.at[pl.ds(bj, W)], srcv[j], isem[j])
        h2 = pltpu.async_copy(dst_hbm.at[pl.ds(bj, W)], dstv[j], isem[j])
        ih.append((h1, h2))
      gh = []
      for j in range(NB):
        ih[j][0].wait()
        ih[j][1].wait()
        gh.append(pltpu.async_copy(h_hbm.at[srcv[j]], rows[j], gsem[j]))
      sh = []
      for j in range(NB):
        gh[j].wait()
        sh.append(pltpu.async_copy(rows[j], acc_sh.at[dstv[j]], ssem[j],
                                   add=True))
      for j in range(NB):
        sh[j].wait()

    tbase = ebase + SS * NB * W
    ih = []
    for t in range(TAIL):
      bt = tbase + t * W
      h1 = pltpu.async_copy(src_hbm.at[pl.ds(bt, W)], srcv[t], isem[t])
      h2 = pltpu.async_copy(dst_hbm.at[pl.ds(bt, W)], dstv[t], isem[t])
      ih.append((h1, h2))
    gh = []
    for t in range(TAIL):
      ih[t][0].wait()
      ih[t][1].wait()
      gh.append(pltpu.async_copy(h_hbm.at[srcv[t]], rows[t], gsem[t]))
    sh = []
    for t in range(TAIL):
      gh[t].wait()
      sh.append(pltpu.async_copy(rows[t], acc_sh.at[dstv[t]], ssem[t],
                                 add=True))
    for t in range(TAIL):
      sh[t].wait()

    plsc.subcore_barrier()
    pltpu.sync_copy(acc_sh.at[pl.ds(row0, RPS)], p_hbm.at[pl.ds(obase, RPS)])

    fill(rows[1], z16)
    zero_my_slice()
    fill(rows[0], o16)
    plsc.subcore_barrier()

    @pl.loop(0, SS)
    def _(s):
      base = ebase + s * NB * W
      ih = []
      for j in range(NB):
        bj = base + j * W
        ih.append(pltpu.async_copy(dst_hbm.at[pl.ds(bj, W)], dstv[j],
                                   isem[j]))
      sh = []
      for j in range(NB):
        ih[j].wait()
        sh.append(pltpu.async_copy(rows[0], acc_sh.at[dstv[j]], ssem[j],
                                   add=True))
      for j in range(NB):
        sh[j].wait()

    ih = []
    for t in range(TAIL):
      bt = tbase + t * W
      ih.append(pltpu.async_copy(dst_hbm.at[pl.ds(bt, W)], dstv[t], isem[t]))
    sh = []
    for t in range(TAIL):
      ih[t].wait()
      sh.append(pltpu.async_copy(rows[0], acc_sh.at[dstv[t]], ssem[t],
                                 add=True))
    for t in range(TAIL):
      sh[t].wait()

    plsc.subcore_barrier()
    pltpu.sync_copy(acc_sh.at[pl.ds(row0, RPS)], c_hbm.at[pl.ds(obase, RPS)])

  p, c = agg_kernel(h, src, dst)
  return p.reshape(NC, NP, D), c.reshape(NC, NP, D)


def _tc_body(h_ref, p_ref, c_ref, wl_ref, bl_ref, wr_ref, ga_ref, be_ref,
             mu_ref, va_ref, o_ref):
  cnt = jnp.maximum(c_ref[0, :, 0:1] + c_ref[1, :, 0:1], 1.0)
  agg = (p_ref[0] + p_ref[1]) / cnt
  hb = h_ref[...]
  dims = (((1,), (1,)), ((), ()))
  out = (lax.dot_general(agg, wl_ref[...], dims,
                         preferred_element_type=jnp.float32)
         + bl_ref[...]
         + lax.dot_general(hb, wr_ref[...], dims,
                           preferred_element_type=jnp.float32))
  s = ga_ref[...] * lax.rsqrt(va_ref[...] + BN_EPS)
  t = be_ref[...] - mu_ref[...] * s
  o_ref[...] = jnp.maximum(out * s + t, 0.0) + hb


def _tc_combine(h, p, c, W_l, b_l, W_r, gamma, beta, mu, var):
  BR = 1024
  full = lambda i: (0, 0)
  return pl.pallas_call(
      _tc_body,
      grid=(NP // BR,),
      in_specs=[
          pl.BlockSpec((BR, D), lambda i: (i, 0)),
          pl.BlockSpec((NC, BR, D), lambda i: (0, i, 0)),
          pl.BlockSpec((NC, BR, D), lambda i: (0, i, 0)),
          pl.BlockSpec((D, D), full),
          pl.BlockSpec((1, D), full),
          pl.BlockSpec((D, D), full),
          pl.BlockSpec((1, D), full),
          pl.BlockSpec((1, D), full),
          pl.BlockSpec((1, D), full),
          pl.BlockSpec((1, D), full),
      ],
      out_specs=pl.BlockSpec((BR, D), lambda i: (i, 0)),
      out_shape=jax.ShapeDtypeStruct((N, D), jnp.float32),
  )(h, p, c, W_l, b_l.reshape(1, D), W_r, gamma.reshape(1, D),
    beta.reshape(1, D), mu.reshape(1, D), var.reshape(1, D))


def kernel(h, edge_index, W_l, b_l, W_r, gamma, beta, running_mean,
           running_var):
  src = edge_index[0]
  dst = edge_index[1]
  p, c = _sc_aggregate(h, src, dst)
  return _tc_combine(h, p, c, W_l, b_l, W_r, gamma, beta, running_mean,
                     running_var)
